# baseline (speedup 1.0000x reference)
import functools

import jax
import jax.numpy as jnp
from jax import lax
from jax.experimental import pallas as pl
from jax.experimental.pallas import tpu as pltpu
from jax.experimental.pallas import tpu_sc as plsc

B = 256
L = 1024
PROJ_DIM = 64
NUM_AA = 20

_TOKENS = B * L

_info = plsc.get_sparse_core_info()
_NC = _info.num_cores
_NS = _info.num_subcores
_NW = _NC * _NS
_ROWS_PER_W = B // _NW
_UNIT = 32768
_NUNIT = _ROWS_PER_W * 2


def _sc_kernel(idx_hbm, w_hbm, b_hbm, out_hbm,
               w_v, b_v, idx_v, buf_a, buf_b, sem_a, sem_b):
    wid = lax.axis_index("s") * _NC + lax.axis_index("c")

    pltpu.sync_copy(w_hbm, w_v)
    pltpu.sync_copy(b_hbm, b_v)
    pltpu.sync_copy(idx_hbm.at[pl.ds(wid * (_ROWS_PER_W * L), _ROWS_PER_W * L)],
                    idx_v)

    bufs = (buf_a, buf_b)
    sems = (sem_a, sem_b)

    def fill(unit, buf):
        b_local = unit >> 1
        h = unit & 1

        def per_p(pp, carry):
            p = h * 32 + pp
            rb_local = pp >> 3
            r = pp & 7
            p_splat = jnp.full((16,), p, jnp.int32)
            bias_vec = plsc.load_gather(b_v, [p_splat])

            def per_cb(cb, c2):
                tok0 = b_local * L + cb * 128
                dst0 = ((rb_local * 8 + cb) * 8 + r) * 128
                for v in range(8):
                    ids = idx_v[pl.ds(tok0 + v * 16, 16)]
                    vals = plsc.load_gather(w_v, [p_splat, ids]) + bias_vec
                    buf[pl.ds(dst0 + v * 16, 16)] = vals
                return c2

            lax.fori_loop(0, 8, per_cb, 0, unroll=False)
            return carry

        lax.fori_loop(0, 32, per_p, 0, unroll=False)

    def out_off(unit):
        return (wid * _ROWS_PER_W + (unit >> 1)) * (2 * _UNIT) \
            + (unit & 1) * _UNIT

    def group(g, carry):
        for parity in range(2):
            unit = g * 2 + parity

            @pl.when(g > 0)
            def _drain():
                pltpu.make_async_copy(
                    bufs[parity], out_hbm.at[pl.ds(0, _UNIT)], sems[parity]
                ).wait()

            fill(unit, bufs[parity])
            pltpu.async_copy(
                bufs[parity], out_hbm.at[pl.ds(out_off(unit), _UNIT)],
                sems[parity],
            )
        return carry

    lax.fori_loop(0, _NUNIT // 2, group, 0, unroll=False)

    for parity in range(2):
        pltpu.make_async_copy(
            bufs[parity], out_hbm.at[pl.ds(0, _UNIT)], sems[parity]
        ).wait()


@jax.jit
def kernel(indices, W, b):
    idx = indices.reshape(_TOKENS).astype(jnp.int32)
    mesh = plsc.VectorSubcoreMesh(core_axis_name="c", subcore_axis_name="s")
    res = pl.kernel(
        _sc_kernel,
        mesh=mesh,
        compiler_params=pltpu.CompilerParams(
            use_tc_tiling_on_sc=False, needs_layout_passes=False
        ),
        out_type=jax.ShapeDtypeStruct((B * PROJ_DIM * L,), jnp.float32),
        scratch_types=[
            pltpu.VMEM((PROJ_DIM, NUM_AA), jnp.float32),
            pltpu.VMEM((PROJ_DIM,), jnp.float32),
            pltpu.VMEM((_ROWS_PER_W * L,), jnp.int32),
            pltpu.VMEM((_UNIT,), jnp.float32),
            pltpu.VMEM((_UNIT,), jnp.float32),
            pltpu.SemaphoreType.DMA,
            pltpu.SemaphoreType.DMA,
        ],
    )(idx, W, b)
    out = (res.reshape(B, 8, 8, 8, 128)
              .transpose(0, 1, 3, 2, 4)
              .reshape(B, PROJ_DIM, L)
              .transpose(0, 2, 1))
    return out
